# baseline (speedup 1.0000x reference)
import functools

import jax
import jax.numpy as jnp
from jax import lax
from jax.experimental import pallas as pl
from jax.experimental.pallas import tpu as pltpu
from jax.experimental.pallas import tpu_sc as plsc

N_NODES = 10000
N_EDGES = 320000
DIM = 128
NUM_REL = 2

N_TILES = 16
CHUNK = 128
CHUNKS_PER_TILE = 160
EDGES_PAD = N_TILES * CHUNKS_PER_TILE * CHUNK
UNROLL = 16
PIPE_ITERS = CHUNKS_PER_TILE // UNROLL
ACC_ROWS = 10240
ZROWS = ACC_ROWS // N_TILES
OROWS = ACC_ROWS // N_TILES


def _sc_body(src_hbm, tgt_hbm, x_hbm, z_hbm, out_hbm,
             srcv, tgtv, rows0, rows1, acc, gs0, gs1, ss0, ss1):
    rows = [rows0, rows1]
    gsems = [gs0, gs1]
    ssems = [ss0, ss1]
    c = lax.axis_index("c")
    t = lax.axis_index("s")

    pltpu.sync_copy(z_hbm, acc.at[pl.ds(t * ZROWS, ZROWS)])
    plsc.subcore_barrier()

    def pipe_step(i, carry):
        pltpu.sync_copy(src_hbm.at[c, t, i], srcv)
        pltpu.sync_copy(tgt_hbm.at[c, t, i], tgtv)
        gathers = [None, None]
        scats = [None, None]
        gathers[0] = pltpu.async_copy(x_hbm.at[srcv.at[0]], rows[0], gs0)
        for b in range(UNROLL):
            rb = b & 1
            gathers[rb].wait()
            if b + 1 < UNROLL:
                if scats[1 - rb] is not None:
                    scats[1 - rb].wait()
                gathers[1 - rb] = pltpu.async_copy(
                    x_hbm.at[srcv.at[b + 1]], rows[1 - rb], gsems[1 - rb])
            scats[rb] = pltpu.async_copy(rows[rb], acc.at[tgtv.at[b]],
                                         ssems[rb], add=True)
        scats[0].wait()
        scats[1].wait()
        return carry

    lax.fori_loop(0, PIPE_ITERS, pipe_step, 0)
    plsc.subcore_barrier()

    pltpu.sync_copy(acc.at[pl.ds(t * OROWS, OROWS)],
                    out_hbm.at[c, pl.ds(t * OROWS, OROWS)])


_sc_scatter = functools.partial(
    pl.kernel,
    mesh=plsc.VectorSubcoreMesh(core_axis_name="c", subcore_axis_name="s"),
    out_type=jax.ShapeDtypeStruct((NUM_REL, ACC_ROWS, DIM), jnp.float32),
    scratch_types=[
        pltpu.VMEM((UNROLL, CHUNK), jnp.int32),
        pltpu.VMEM((UNROLL, CHUNK), jnp.int32),
        pltpu.VMEM((CHUNK, DIM), jnp.float32),
        pltpu.VMEM((CHUNK, DIM), jnp.float32),
        pltpu.VMEM_SHARED((ACC_ROWS, DIM), jnp.float32),
        pltpu.SemaphoreType.DMA,
        pltpu.SemaphoreType.DMA,
        pltpu.SemaphoreType.DMA,
        pltpu.SemaphoreType.DMA,
    ],
)(_sc_body)


def _tc_body(x_ref, parts_ref, w_ref, o_ref):
    dn = (((1,), (1,)), ((), ()))
    o = lax.dot_general(x_ref[...], w_ref[0], dn,
                        preferred_element_type=jnp.float32)
    o += lax.dot_general(parts_ref[0], w_ref[1], dn,
                         preferred_element_type=jnp.float32)
    o += lax.dot_general(parts_ref[1], w_ref[2], dn,
                         preferred_element_type=jnp.float32)
    o_ref[...] = o


_TC_BLOCK = 1000


def kernel(x, edge_indices, W0, W1, W_self):
    src = edge_indices[:, 0, :]
    tgt = edge_indices[:, 1, :]
    pad = EDGES_PAD - N_EDGES
    pad_tgt = N_NODES + (jnp.arange(pad, dtype=jnp.int32) % (ACC_ROWS - N_NODES))
    pad_src = jnp.arange(pad, dtype=jnp.int32) % N_NODES
    src_p = jnp.concatenate(
        [src, jnp.broadcast_to(pad_src, (NUM_REL, pad))], axis=1
    ).reshape(NUM_REL, N_TILES, PIPE_ITERS, UNROLL, CHUNK)
    tgt_p = jnp.concatenate(
        [tgt, jnp.broadcast_to(pad_tgt, (NUM_REL, pad))], axis=1
    ).reshape(NUM_REL, N_TILES, PIPE_ITERS, UNROLL, CHUNK)
    zeros_blk = jnp.zeros((ZROWS, DIM), jnp.float32)

    parts = _sc_scatter(src_p, tgt_p, x, zeros_blk)

    w = jnp.stack([W_self, W0, W1])
    grid = (N_NODES // _TC_BLOCK,)
    out = pl.pallas_call(
        _tc_body,
        grid=grid,
        in_specs=[
            pl.BlockSpec((_TC_BLOCK, DIM), lambda i: (i, 0)),
            pl.BlockSpec((NUM_REL, _TC_BLOCK, DIM), lambda i: (0, i, 0)),
            pl.BlockSpec((3, DIM, DIM), lambda i: (0, 0, 0)),
        ],
        out_specs=pl.BlockSpec((_TC_BLOCK, DIM), lambda i: (i, 0)),
        out_shape=jax.ShapeDtypeStruct((N_NODES, DIM), jnp.float32),
    )(x, parts, w)
    return out


# speedup vs baseline: 1.1868x; 1.1868x over previous
import functools

import jax
import jax.numpy as jnp
from jax import lax
from jax.experimental import pallas as pl
from jax.experimental.pallas import tpu as pltpu
from jax.experimental.pallas import tpu_sc as plsc

N_NODES = 10000
N_EDGES = 320000
DIM = 128
NUM_REL = 2

N_TILES = 16
CHUNK = 128
CHUNKS_PER_TILE = 160
EDGES_PAD = N_TILES * CHUNKS_PER_TILE * CHUNK
UNROLL = 32
PIPE_ITERS = CHUNKS_PER_TILE // UNROLL
ACC_ROWS = 10240
ZROWS = ACC_ROWS // N_TILES
OROWS = ACC_ROWS // N_TILES


def _sc_body(src_hbm, tgt_hbm, x_hbm, z_hbm, out_hbm,
             srcv, tgtv, rows0, rows1, acc, gs0, gs1):
    rows = [rows0, rows1]
    gsems = [gs0, gs1]
    c = lax.axis_index("c")
    t = lax.axis_index("s")

    pltpu.sync_copy(z_hbm, acc.at[pl.ds(t * ZROWS, ZROWS)])
    plsc.subcore_barrier()

    def pipe_step(i, carry):
        pltpu.sync_copy(src_hbm.at[c, t, i], srcv)
        pltpu.sync_copy(tgt_hbm.at[c, t, i], tgtv)
        gathers = [None, None]
        gathers[0] = pltpu.async_copy(x_hbm.at[srcv.at[0]], rows[0], gs0)
        for b in range(UNROLL):
            rb = b & 1
            if b + 1 < UNROLL:
                gathers[1 - rb] = pltpu.async_copy(
                    x_hbm.at[srcv.at[b + 1]], rows[1 - rb], gsems[1 - rb])
            gathers[rb].wait()
            pltpu.sync_copy(rows[rb], acc.at[tgtv.at[b]], add=True)
        return carry

    lax.fori_loop(0, PIPE_ITERS, pipe_step, 0)
    plsc.subcore_barrier()

    pltpu.sync_copy(acc.at[pl.ds(t * OROWS, OROWS)],
                    out_hbm.at[c, pl.ds(t * OROWS, OROWS)])


_sc_scatter = functools.partial(
    pl.kernel,
    mesh=plsc.VectorSubcoreMesh(core_axis_name="c", subcore_axis_name="s"),
    out_type=jax.ShapeDtypeStruct((NUM_REL, ACC_ROWS, DIM), jnp.float32),
    scratch_types=[
        pltpu.VMEM((UNROLL, CHUNK), jnp.int32),
        pltpu.VMEM((UNROLL, CHUNK), jnp.int32),
        pltpu.VMEM((CHUNK, DIM), jnp.float32),
        pltpu.VMEM((CHUNK, DIM), jnp.float32),
        pltpu.VMEM_SHARED((ACC_ROWS, DIM), jnp.float32),
        pltpu.SemaphoreType.DMA,
        pltpu.SemaphoreType.DMA,
    ],
)(_sc_body)


def _tc_body(x_ref, parts_ref, w_ref, o_ref):
    dn = (((1,), (1,)), ((), ()))
    o = lax.dot_general(x_ref[...], w_ref[0], dn,
                        preferred_element_type=jnp.float32)
    o += lax.dot_general(parts_ref[0], w_ref[1], dn,
                         preferred_element_type=jnp.float32)
    o += lax.dot_general(parts_ref[1], w_ref[2], dn,
                         preferred_element_type=jnp.float32)
    o_ref[...] = o


_TC_BLOCK = 1000


def kernel(x, edge_indices, W0, W1, W_self):
    src = edge_indices[:, 0, :]
    tgt = edge_indices[:, 1, :]
    pad = EDGES_PAD - N_EDGES
    pad_tgt = N_NODES + (jnp.arange(pad, dtype=jnp.int32) % (ACC_ROWS - N_NODES))
    pad_src = jnp.arange(pad, dtype=jnp.int32) % N_NODES
    src_p = jnp.concatenate(
        [src, jnp.broadcast_to(pad_src, (NUM_REL, pad))], axis=1
    ).reshape(NUM_REL, N_TILES, PIPE_ITERS, UNROLL, CHUNK)
    tgt_p = jnp.concatenate(
        [tgt, jnp.broadcast_to(pad_tgt, (NUM_REL, pad))], axis=1
    ).reshape(NUM_REL, N_TILES, PIPE_ITERS, UNROLL, CHUNK)
    zeros_blk = jnp.zeros((ZROWS, DIM), jnp.float32)

    parts = _sc_scatter(src_p, tgt_p, x, zeros_blk)

    w = jnp.stack([W_self, W0, W1])
    grid = (N_NODES // _TC_BLOCK,)
    out = pl.pallas_call(
        _tc_body,
        grid=grid,
        in_specs=[
            pl.BlockSpec((_TC_BLOCK, DIM), lambda i: (i, 0)),
            pl.BlockSpec((NUM_REL, _TC_BLOCK, DIM), lambda i: (0, i, 0)),
            pl.BlockSpec((3, DIM, DIM), lambda i: (0, 0, 0)),
        ],
        out_specs=pl.BlockSpec((_TC_BLOCK, DIM), lambda i: (i, 0)),
        out_shape=jax.ShapeDtypeStruct((N_NODES, DIM), jnp.float32),
    )(x, parts, w)
    return out
